# baseline (speedup 1.0000x reference)
import jax
import jax.numpy as jnp
from jax import lax
from jax.experimental import pallas as pl
from jax.experimental.pallas import tpu as pltpu

_B, _C, _H, _W = 8, 96, 384, 384
_K = 8
_CB = 48
_NJ = _C // _CB


def _fused_body(x_ref, w1t_ref, w2t_ref, idx_ref, sums_ref):
    b = pl.program_id(0)
    j = pl.program_id(1)
    sums_ref[b * _NJ + j] = jnp.sum(x_ref[...], axis=(2, 3))

    @pl.when(jnp.logical_and(b == _B - 1, j == _NJ - 1))
    def _():
        rows = []
        for bb in range(_B):
            parts = [sums_ref[bb * _NJ + jj] for jj in range(_NJ)]
            rows.append(jnp.concatenate(parts, axis=1))
        y = jnp.concatenate(rows, axis=0) * (1.0 / (_H * _W))
        h = jnp.maximum(
            jnp.dot(y, w1t_ref[...], preferred_element_type=jnp.float32), 0.0
        )
        z = jnp.dot(h, w2t_ref[...], preferred_element_type=jnp.float32)
        z = jax.nn.sigmoid(z)
        iota = lax.broadcasted_iota(jnp.int32, (_B, _C), 1)
        cols = []
        for _ in range(_K):
            mx = jnp.max(z, axis=1, keepdims=True)
            idt = jnp.min(jnp.where(z == mx, iota, _C), axis=1)
            cols.append(idt)
            z = jnp.where(iota == idt[:, None], -1.0, z)
        idx_ref[...] = jnp.stack(cols, axis=1).astype(jnp.int32)


_G = 8


def _gather_body(idx_ref, *refs):
    x_refs, o_ref = refs[:_G], refs[_G]
    for t in range(_G):
        o_ref[0, t] = x_refs[t][0, 0]


def kernel(x, W1, W2):
    b, c, h, w = x.shape

    idx = pl.pallas_call(
        _fused_body,
        grid=(_B, _NJ),
        in_specs=[
            pl.BlockSpec((1, _CB, _H, _W), lambda b, j: (b, j, 0, 0)),
            pl.BlockSpec((_C, _C), lambda b, j: (0, 0)),
            pl.BlockSpec((_C, _C), lambda b, j: (0, 0)),
        ],
        out_specs=pl.BlockSpec((_B, _K), lambda b, j: (0, 0)),
        out_shape=jax.ShapeDtypeStruct((_B, _K), jnp.int32),
        scratch_shapes=[pltpu.VMEM((_B * _NJ, 1, _CB), jnp.float32)],
    )(x, W1.T, W2.T)

    idx_flat = idx.reshape(_B * _K)

    out = pl.pallas_call(
        _gather_body,
        grid_spec=pltpu.PrefetchScalarGridSpec(
            num_scalar_prefetch=1,
            grid=(_B * _K // _G,),
            in_specs=[
                pl.BlockSpec(
                    (1, 1, _H, _W),
                    lambda i, idx_ref, t=t: (i // (_K // _G), idx_ref[i * _G + t], 0, 0),
                )
                for t in range(_G)
            ],
            out_specs=pl.BlockSpec(
                (1, _G, _H, _W),
                lambda i, idx_ref: (i // (_K // _G), i % (_K // _G), 0, 0),
            ),
        ),
        out_shape=jax.ShapeDtypeStruct((_B, _K, _H, _W), jnp.float32),
    )(idx_flat, *([x] * _G))

    return out


# speedup vs baseline: 1.0036x; 1.0036x over previous
import jax
import jax.numpy as jnp
from jax import lax
from jax.experimental import pallas as pl
from jax.experimental.pallas import tpu as pltpu

_B, _C, _H, _W = 8, 96, 384, 384
_K = 8
_CB = 32
_NJ = _C // _CB


def _fused_body(x_ref, w1t_ref, w2t_ref, idx_ref, sums_ref):
    b = pl.program_id(0)
    j = pl.program_id(1)
    sums_ref[b * _NJ + j] = jnp.sum(x_ref[...], axis=(2, 3))

    @pl.when(jnp.logical_and(b == _B - 1, j == _NJ - 1))
    def _():
        rows = []
        for bb in range(_B):
            parts = [sums_ref[bb * _NJ + jj] for jj in range(_NJ)]
            rows.append(jnp.concatenate(parts, axis=1))
        y = jnp.concatenate(rows, axis=0) * (1.0 / (_H * _W))
        h = jnp.maximum(
            jnp.dot(y, w1t_ref[...], preferred_element_type=jnp.float32), 0.0
        )
        z = jnp.dot(h, w2t_ref[...], preferred_element_type=jnp.float32)
        z = jax.nn.sigmoid(z)
        iota = lax.broadcasted_iota(jnp.int32, (_B, _C), 1)
        cols = []
        for _ in range(_K):
            mx = jnp.max(z, axis=1, keepdims=True)
            idt = jnp.min(jnp.where(z == mx, iota, _C), axis=1)
            cols.append(idt)
            z = jnp.where(iota == idt[:, None], -1.0, z)
        idx_ref[...] = jnp.stack(cols, axis=1).astype(jnp.int32)


_G = 8


def _gather_body(idx_ref, *refs):
    x_refs, o_ref = refs[:_G], refs[_G]
    for t in range(_G):
        o_ref[0, t] = x_refs[t][0, 0]


def kernel(x, W1, W2):
    b, c, h, w = x.shape

    idx = pl.pallas_call(
        _fused_body,
        grid=(_B, _NJ),
        in_specs=[
            pl.BlockSpec((1, _CB, _H, _W), lambda b, j: (b, j, 0, 0)),
            pl.BlockSpec((_C, _C), lambda b, j: (0, 0)),
            pl.BlockSpec((_C, _C), lambda b, j: (0, 0)),
        ],
        out_specs=pl.BlockSpec((_B, _K), lambda b, j: (0, 0)),
        out_shape=jax.ShapeDtypeStruct((_B, _K), jnp.int32),
        scratch_shapes=[pltpu.VMEM((_B * _NJ, 1, _CB), jnp.float32)],
    )(x, W1.T, W2.T)

    idx_flat = idx.reshape(_B * _K)

    out = pl.pallas_call(
        _gather_body,
        grid_spec=pltpu.PrefetchScalarGridSpec(
            num_scalar_prefetch=1,
            grid=(_B * _K // _G,),
            in_specs=[
                pl.BlockSpec(
                    (1, 1, _H, _W),
                    lambda i, idx_ref, t=t: (i // (_K // _G), idx_ref[i * _G + t], 0, 0),
                )
                for t in range(_G)
            ],
            out_specs=pl.BlockSpec(
                (1, _G, _H, _W),
                lambda i, idx_ref: (i // (_K // _G), i % (_K // _G), 0, 0),
            ),
        ),
        out_shape=jax.ShapeDtypeStruct((_B, _K, _H, _W), jnp.float32),
    )(idx_flat, *([x] * _G))

    return out
